# Initial kernel scaffold; baseline (speedup 1.0000x reference)
import jax
import jax.numpy as jnp
from jax.experimental import pallas as pl


def kernel(x, edge_index, edge_attr, batch, W1, b1, W2, b2, W3, b3, fcW1, fcb1, fcW2, fcb2):
    raise NotImplementedError("write your pallas kernel here")


# speedup vs baseline: 16.9816x; 16.9816x over previous
import functools

import jax
import jax.numpy as jnp
from jax import lax
from jax.experimental import pallas as pl
from jax.experimental.pallas import tpu as pltpu
from jax.experimental.pallas import tpu_sc as plsc

N = 10000
E = 320000
F = 128
G = 64
NC = 2
NS = 16
NW = NC * NS
EPW = E // NW
CH = 80
NCHUNK = EPW // CH
SLAB = 624
REM = N - NS * SLAB
BLK = 1000
GRID = N // BLK

_mesh = plsc.VectorSubcoreMesh(core_axis_name="c", subcore_axis_name="s")


@functools.partial(
    pl.kernel,
    out_type=jax.ShapeDtypeStruct((NW, N), jnp.float32),
    mesh=_mesh,
    scratch_types=[
        pltpu.VMEM((EPW,), jnp.int32),
        pltpu.VMEM((N,), jnp.float32),
    ],
    compiler_params=pltpu.CompilerParams(needs_layout_passes=False),
)
def _sc_deg(dst_hbm, deg_out, dst_v, deg_v):
    c = lax.axis_index("c")
    s = lax.axis_index("s")
    wid = s * NC + c

    def zero_row(i, carry):
        deg_v[pl.ds(i * 16, 16)] = jnp.zeros((16,), jnp.float32)
        return carry

    lax.fori_loop(0, N // 16, zero_row, 0)

    pltpu.sync_copy(dst_hbm.at[wid], dst_v)
    ones = jnp.ones((16,), jnp.float32)

    def body(j, carry):
        idx = dst_v[pl.ds(j * 16, 16)]
        plsc.addupdate_scatter(deg_v, [idx], ones)
        return carry

    lax.fori_loop(0, EPW // 16, body, 0)
    pltpu.sync_copy(deg_v, deg_out.at[wid])


@functools.partial(
    pl.kernel,
    out_type=jax.ShapeDtypeStruct((NC, N, F), jnp.float32),
    mesh=_mesh,
    scratch_types=[
        pltpu.VMEM((NCHUNK, CH), jnp.int32),
        pltpu.VMEM((NCHUNK, CH), jnp.int32),
        pltpu.VMEM((CH, F), jnp.float32),
        pltpu.SemaphoreType.DMA,
        pltpu.VMEM_SHARED((N, F), jnp.float32),
    ],
    compiler_params=pltpu.CompilerParams(needs_layout_passes=False),
)
def _sc_agg(h_hbm, src_hbm, dst_hbm, zero_hbm, p_out, src_v, dst_v, rows_v,
            sem, acc):
    c = lax.axis_index("c")
    s = lax.axis_index("s")
    wid = s * NC + c

    pltpu.sync_copy(zero_hbm.at[pl.ds(0, SLAB)], acc.at[pl.ds(s * SLAB, SLAB)])

    @pl.when(s == 0)
    def _():
        pltpu.sync_copy(zero_hbm.at[pl.ds(0, REM)],
                        acc.at[pl.ds(NS * SLAB, REM)])

    pltpu.sync_copy(src_hbm.at[wid], src_v)
    pltpu.sync_copy(dst_hbm.at[wid], dst_v)
    plsc.subcore_barrier()

    def body(j, carry):
        pltpu.async_copy(h_hbm.at[src_v.at[j]], rows_v, sem).wait()
        pltpu.sync_copy(rows_v, acc.at[dst_v.at[j]], add=True)
        return carry

    lax.fori_loop(0, NCHUNK, body, 0)

    plsc.subcore_barrier()
    pltpu.sync_copy(acc.at[pl.ds(s * SLAB, SLAB)],
                    p_out.at[c, pl.ds(s * SLAB, SLAB)])

    @pl.when(s == 0)
    def _():
        pltpu.sync_copy(acc.at[pl.ds(NS * SLAB, REM)],
                        p_out.at[c, pl.ds(NS * SLAB, REM)])


def _tc0_body(degp_ref, x_ref, w_ref, dinv_ref, hw_ref):
    deg = jnp.sum(degp_ref[0], axis=0) + 1.0
    dinv = lax.rsqrt(deg)
    h = jnp.dot(x_ref[...], w_ref[...], preferred_element_type=jnp.float32)
    hw_ref[...] = h * dinv[:, None]
    dinv_ref[...] = dinv[:, None]


def _tc_mid_body(p_ref, hwp_ref, dinv_ref, b_ref, w_ref, out_ref):
    dinv = dinv_ref[...]
    tot = p_ref[0] + p_ref[1] + hwp_ref[...]
    a = jnp.maximum(tot * dinv + b_ref[...], 0.0)
    out_ref[...] = jnp.dot(
        a, w_ref[...], preferred_element_type=jnp.float32) * dinv


def _tc_pool_body(p_ref, hwp_ref, dinv_ref, b_ref, batch_ref, psum_ref,
                  cnt_ref):
    j = pl.program_id(0)
    tot = p_ref[0] + p_ref[1] + hwp_ref[...]
    h3 = jnp.maximum(tot * dinv_ref[...] + b_ref[...], 0.0)
    bt = batch_ref[0, 0]
    ind = (bt[None, :] == lax.broadcasted_iota(jnp.int32, (G, BLK), 0)
           ).astype(jnp.float32)
    ps = jnp.dot(ind, h3, preferred_element_type=jnp.float32)
    cs = jnp.sum(ind, axis=1, keepdims=True)

    @pl.when(j == 0)
    def _():
        psum_ref[...] = ps
        cnt_ref[...] = cs

    @pl.when(j > 0)
    def _():
        psum_ref[...] += ps
        cnt_ref[...] += cs


def _tc_fin_body(ps_ref, cnt_ref, w1_ref, b1_ref, w2_ref, b2_ref, out_ref):
    pooled = ps_ref[...] / jnp.maximum(cnt_ref[...], 1.0)
    r = jnp.maximum(
        jnp.dot(pooled, w1_ref[...], preferred_element_type=jnp.float32)
        + b1_ref[...], 0.0)
    out_ref[...] = jnp.dot(
        r, w2_ref[...], preferred_element_type=jnp.float32) + b2_ref[...]


def _tc0(deg_p, x, W1):
    return pl.pallas_call(
        _tc0_body,
        grid=(GRID,),
        in_specs=[
            pl.BlockSpec((1, NW, BLK), lambda j: (j, 0, 0)),
            pl.BlockSpec((BLK, F), lambda j: (j, 0)),
            pl.BlockSpec((F, F), lambda j: (0, 0)),
        ],
        out_specs=[
            pl.BlockSpec((BLK, 1), lambda j: (j, 0)),
            pl.BlockSpec((BLK, F), lambda j: (j, 0)),
        ],
        out_shape=[
            jax.ShapeDtypeStruct((N, 1), jnp.float32),
            jax.ShapeDtypeStruct((N, F), jnp.float32),
        ],
    )(deg_p, x, W1)


def _tc_mid(p, hwp, dinv, b, Wn):
    return pl.pallas_call(
        _tc_mid_body,
        grid=(GRID,),
        in_specs=[
            pl.BlockSpec((NC, BLK, F), lambda j: (0, j, 0)),
            pl.BlockSpec((BLK, F), lambda j: (j, 0)),
            pl.BlockSpec((BLK, 1), lambda j: (j, 0)),
            pl.BlockSpec((1, F), lambda j: (0, 0)),
            pl.BlockSpec((F, F), lambda j: (0, 0)),
        ],
        out_specs=pl.BlockSpec((BLK, F), lambda j: (j, 0)),
        out_shape=jax.ShapeDtypeStruct((N, F), jnp.float32),
    )(p, hwp, dinv, b.reshape(1, F), Wn)


def _tc_pool(p, hwp, dinv, b, batch2):
    return pl.pallas_call(
        _tc_pool_body,
        grid=(GRID,),
        in_specs=[
            pl.BlockSpec((NC, BLK, F), lambda j: (0, j, 0)),
            pl.BlockSpec((BLK, F), lambda j: (j, 0)),
            pl.BlockSpec((BLK, 1), lambda j: (j, 0)),
            pl.BlockSpec((1, F), lambda j: (0, 0)),
            pl.BlockSpec((1, 1, BLK), lambda j: (j, 0, 0)),
        ],
        out_specs=[
            pl.BlockSpec((G, F), lambda j: (0, 0)),
            pl.BlockSpec((G, 1), lambda j: (0, 0)),
        ],
        out_shape=[
            jax.ShapeDtypeStruct((G, F), jnp.float32),
            jax.ShapeDtypeStruct((G, 1), jnp.float32),
        ],
    )(p, hwp, dinv, b.reshape(1, F), batch2)


def _tc_fin(psum, cnt, fcW1, fcb1, fcW2, fcb2):
    return pl.pallas_call(
        _tc_fin_body,
        in_specs=[
            pl.BlockSpec((G, F), lambda: (0, 0)),
            pl.BlockSpec((G, 1), lambda: (0, 0)),
            pl.BlockSpec((F, 32), lambda: (0, 0)),
            pl.BlockSpec((1, 32), lambda: (0, 0)),
            pl.BlockSpec((32, 1), lambda: (0, 0)),
            pl.BlockSpec((1, 1), lambda: (0, 0)),
        ],
        out_specs=pl.BlockSpec((G, 1), lambda: (0, 0)),
        out_shape=jax.ShapeDtypeStruct((G, 1), jnp.float32),
    )(psum, cnt, fcW1, fcb1.reshape(1, 32), fcW2, fcb2.reshape(1, 1))


def kernel(x, edge_index, edge_attr, batch, W1, b1, W2, b2, W3, b3,
           fcW1, fcb1, fcW2, fcb2):
    src_r = edge_index[0].reshape(NW, NCHUNK, CH)
    dst_r = edge_index[1].reshape(NW, NCHUNK, CH)
    dst_flat = edge_index[1].reshape(NW, EPW)
    zero_slab = jnp.zeros((SLAB, F), jnp.float32)

    deg_p = _sc_deg(dst_flat)
    deg_pt = deg_p.reshape(NW, GRID, BLK).transpose(1, 0, 2)
    dinv, hw1 = _tc0(deg_pt, x, W1)

    p1 = _sc_agg(hw1, src_r, dst_r, zero_slab)
    hw2 = _tc_mid(p1, hw1, dinv, b1, W2)

    p2 = _sc_agg(hw2, src_r, dst_r, zero_slab)
    hw3 = _tc_mid(p2, hw2, dinv, b2, W3)

    p3 = _sc_agg(hw3, src_r, dst_r, zero_slab)
    psum, cnt = _tc_pool(p3, hw3, dinv, b3, batch.reshape(GRID, 1, BLK))

    return _tc_fin(psum, cnt, fcW1, fcb1, fcW2, fcb2)
